# Initial kernel scaffold; baseline (speedup 1.0000x reference)
import jax
import jax.numpy as jnp
from jax.experimental import pallas as pl


def kernel(x, edge_index, params):
    raise NotImplementedError("write your pallas kernel here")


# speedup vs baseline: 4.4260x; 4.4260x over previous
import functools

import jax
import jax.numpy as jnp
from jax import lax
from jax.experimental import pallas as pl
from jax.experimental.pallas import tpu as pltpu
from jax.experimental.pallas import tpu_sc as plsc

N_NODES = 10000
IN_DIM = 128
HID = 64
BN_EPS = 1e-5

NW = 32
CH = 128
CHUNKS = 80
E_PAD = NW * CHUNKS * CH
N_PAD = 10112
RPT = N_PAD // 16

_sc_mesh = plsc.VectorSubcoreMesh(core_axis_name="c", subcore_axis_name="s")


@functools.partial(
    pl.kernel,
    mesh=_sc_mesh,
    compiler_params=pltpu.CompilerParams(use_tc_tiling_on_sc=False),
    out_type=jax.ShapeDtypeStruct((2, N_PAD, HID), jnp.float32),
    scratch_types=[
        pltpu.VMEM((CHUNKS, CH), jnp.int32),
        pltpu.VMEM((CHUNKS, CH), jnp.int32),
        pltpu.VMEM((CH, HID), jnp.float32),
        pltpu.VMEM((RPT, HID), jnp.float32),
        pltpu.VMEM_SHARED((N_PAD, HID), jnp.float32),
        pltpu.SemaphoreType.DMA,
    ],
)
def _sc_agg(y_hbm, col_hbm, row_hbm, out_hbm,
            col_v, row_v, rows_v, stage_v, acc_sh, sem):
    c = lax.axis_index("c")
    s = lax.axis_index("s")
    wid = s * 2 + c

    zero16 = jnp.zeros((16,), jnp.float32)

    def _zero_row(i, carry):
        for j in range(HID // 16):
            stage_v[i, pl.ds(j * 16, 16)] = zero16
        return carry

    lax.fori_loop(0, RPT, _zero_row, 0)
    pltpu.sync_copy(stage_v, acc_sh.at[pl.ds(s * RPT, RPT)])

    pltpu.sync_copy(col_hbm.at[wid], col_v)
    pltpu.sync_copy(row_hbm.at[wid], row_v)
    plsc.subcore_barrier()

    def _step(j, carry):
        pltpu.async_copy(y_hbm.at[col_v.at[j]], rows_v, sem).wait()
        pltpu.sync_copy(rows_v, acc_sh.at[row_v.at[j]], add=True)
        return carry

    lax.fori_loop(0, CHUNKS, _step, 0)
    plsc.subcore_barrier()

    pltpu.sync_copy(acc_sh.at[pl.ds(s * RPT, RPT)], stage_v)
    pltpu.sync_copy(stage_v, out_hbm.at[c, pl.ds(s * RPT, RPT)])


def _proj_body(x_ref, w_ref, o_ref):
    o_ref[...] = jnp.dot(x_ref[...], w_ref[...],
                         preferred_element_type=jnp.float32)


def _combine_body(y_ref, p0_ref, p1_ref, srow_ref, c1_ref, w2_ref, c2_ref,
                  w1n_ref, o_ref):
    t = y_ref[...] * srow_ref[...] + p0_ref[...] + p1_ref[...] + c1_ref[...]
    t = jnp.maximum(t, 0.0)
    u = jnp.dot(t, w2_ref[...], preferred_element_type=jnp.float32)
    u = jnp.maximum(u + c2_ref[...], 0.0)
    o_ref[...] = jnp.dot(u, w1n_ref[...], preferred_element_type=jnp.float32)


def _combine_last_body(y_ref, p0_ref, p1_ref, srow_ref, c1_ref, w2_ref,
                       c2_ref, o_ref):
    t = y_ref[...] * srow_ref[...] + p0_ref[...] + p1_ref[...] + c1_ref[...]
    t = jnp.maximum(t, 0.0)
    u = jnp.dot(t, w2_ref[...], preferred_element_type=jnp.float32)
    o_ref[...] = u + c2_ref[...]


_BR = 2000
_GRID = N_NODES // _BR


def _row_spec(d):
    return pl.BlockSpec((_BR, d), lambda i: (i, 0))


def _full_spec(r, d):
    return pl.BlockSpec((r, d), lambda i: (0, 0))


def _tc_proj(x, w):
    return pl.pallas_call(
        _proj_body,
        grid=(_GRID,),
        in_specs=[_row_spec(IN_DIM), _full_spec(IN_DIM, HID)],
        out_specs=_row_spec(HID),
        out_shape=jax.ShapeDtypeStruct((N_NODES, HID), jnp.float32),
    )(x, w)


def _tc_combine(y, p0, p1, srow, c1, w2, c2, w1n):
    return pl.pallas_call(
        _combine_body,
        grid=(_GRID,),
        in_specs=[_row_spec(HID), _row_spec(HID), _row_spec(HID),
                  _full_spec(1, HID), _full_spec(1, HID),
                  _full_spec(HID, HID), _full_spec(1, HID),
                  _full_spec(HID, HID)],
        out_specs=_row_spec(HID),
        out_shape=jax.ShapeDtypeStruct((N_NODES, HID), jnp.float32),
    )(y, p0, p1, srow, c1, w2, c2, w1n)


def _tc_combine_last(y, p0, p1, srow, c1, w2, c2):
    return pl.pallas_call(
        _combine_last_body,
        grid=(_GRID,),
        in_specs=[_row_spec(HID), _row_spec(HID), _row_spec(HID),
                  _full_spec(1, HID), _full_spec(1, HID),
                  _full_spec(HID, HID), _full_spec(1, HID)],
        out_specs=_row_spec(HID),
        out_shape=jax.ShapeDtypeStruct((N_NODES, HID), jnp.float32),
    )(y, p0, p1, srow, c1, w2, c2)


def kernel(x, edge_index, params):
    inv = 1.0 / jnp.sqrt(1.0 + BN_EPS)
    folded = []
    for p in params:
        a1 = p["g1"] * inv
        a2 = p["g2"] * inv
        folded.append({
            "W1": p["W1"] * a1[None, :],
            "c1": (p["b1"] * a1 + p["be1"])[None, :],
            "W2": p["W2"] * a2[None, :],
            "c2": (p["b2"] * a2 + p["be2"])[None, :],
            "s": ((1.0 + p["eps"][0]) * jnp.ones((HID,), jnp.float32))[None, :],
        })

    row = edge_index[0]
    col = edge_index[1]
    pad = E_PAD - row.shape[0]
    row3 = jnp.concatenate(
        [row, jnp.full((pad,), N_NODES, jnp.int32)]).reshape(NW, CHUNKS, CH)
    col3 = jnp.concatenate(
        [col, jnp.zeros((pad,), jnp.int32)]).reshape(NW, CHUNKS, CH)

    y = _tc_proj(x, folded[0]["W1"])
    for i, f in enumerate(folded):
        parts = _sc_agg(y, col3, row3)
        p0 = parts[0, :N_NODES]
        p1 = parts[1, :N_NODES]
        if i < len(folded) - 1:
            y = _tc_combine(y, p0, p1, f["s"], f["c1"], f["W2"], f["c2"],
                            folded[i + 1]["W1"])
        else:
            y = _tc_combine_last(y, p0, p1, f["s"], f["c1"], f["W2"], f["c2"])
    return y
